# Initial kernel scaffold; baseline (speedup 1.0000x reference)
import jax
import jax.numpy as jnp
from jax.experimental import pallas as pl


def kernel(x, edge_index, W1, b1, ln_gamma, ln_beta, prelu_a, W2, b2):
    raise NotImplementedError("write your pallas kernel here")


# speedup vs baseline: 4.9608x; 4.9608x over previous
import functools

import jax
import jax.numpy as jnp
from jax import lax
from jax.experimental import pallas as pl
from jax.experimental.pallas import tpu as pltpu
from jax.experimental.pallas import tpu_sc as plsc

NC = 2
NS = 16
NW = NC * NS
CHUNK = 128


def _pad_rows(n):
    per_tile = -(-n // (NS * CHUNK)) * CHUNK
    return NS * per_tile


def _mesh():
    return plsc.VectorSubcoreMesh(core_axis_name="c", subcore_axis_name="s")


def _zero_rows(buf, rows, width):
    @pl.loop(0, rows)
    def _(i):
        for j in range(width // 16):
            buf[i, pl.ds(j * 16, 16)] = jnp.zeros((16,), jnp.float32)


def _copy_rows_spmem(src_buf, dst_sh, base, rows):
    for b in range(rows // CHUNK):
        pltpu.sync_copy(src_buf, dst_sh.at[pl.ds(base + b * CHUNK, CHUNK)])


def _copy_out_spmem(src_sh, out_hbm, cid, base, rows):
    for b in range(rows // CHUNK):
        pltpu.sync_copy(src_sh.at[pl.ds(base + b * CHUNK, CHUNK)],
                        out_hbm.at[cid, pl.ds(base + b * CHUNK, CHUNK)])


@functools.lru_cache(maxsize=None)
def _make_degree_kernel(n, e):
    e_per_tile = e // NW
    n_full = e_per_tile // CHUNK
    tail = e_per_tile - n_full * CHUNK
    npad = _pad_rows(n)
    hrows = npad // CHUNK

    @functools.partial(
        pl.kernel,
        mesh=_mesh(),
        compiler_params=pltpu.CompilerParams(needs_layout_passes=False),
        out_type=(jax.ShapeDtypeStruct((NW * npad,), jnp.float32),
                  jax.ShapeDtypeStruct((NW * npad,), jnp.float32)),
        scratch_types=[
            pltpu.VMEM((CHUNK,), jnp.int32),
            pltpu.VMEM((CHUNK,), jnp.int32),
            pltpu.VMEM((npad,), jnp.float32),
            pltpu.VMEM((npad,), jnp.float32),
        ],
    )
    def k(src_hbm, dst_hbm, outs_hbm, outd_hbm,
          src_v, dst_v, cnts_v, cntd_v):
        cid = lax.axis_index("c")
        sid = lax.axis_index("s")
        wid = sid * NC + cid

        @pl.loop(0, npad // 16)
        def _(i):
            cnts_v[pl.ds(i * 16, 16)] = jnp.zeros((16,), jnp.float32)
            cntd_v[pl.ds(i * 16, 16)] = jnp.zeros((16,), jnp.float32)

        eb = wid * e_per_tile
        ones16 = jnp.ones((16,), jnp.float32)

        def count_chunk(off, width):
            pltpu.sync_copy(src_hbm.at[pl.ds(off, width)],
                            src_v.at[pl.ds(0, width)])
            pltpu.sync_copy(dst_hbm.at[pl.ds(off, width)],
                            dst_v.at[pl.ds(0, width)])
            for j in range(width // 16):
                plsc.addupdate_scatter(
                    cnts_v, [src_v[pl.ds(j * 16, 16)]], ones16)
                plsc.addupdate_scatter(
                    cntd_v, [dst_v[pl.ds(j * 16, 16)]], ones16)

        @pl.loop(0, n_full)
        def _(c):
            count_chunk(eb + c * CHUNK, CHUNK)

        if tail:
            count_chunk(eb + n_full * CHUNK, tail)

        pltpu.sync_copy(cnts_v, outs_hbm.at[pl.ds(wid * npad, npad)])
        pltpu.sync_copy(cntd_v, outd_hbm.at[pl.ds(wid * npad, npad)])

    return k


@functools.lru_cache(maxsize=None)
def _make_segsum_kernel(n, e, d):
    e_per_core = e // NC
    e_per_tile = e_per_core // NS
    n_full = e_per_tile // CHUNK
    tail = e_per_tile - n_full * CHUNK
    npad = _pad_rows(n)
    rows_per_tile = npad // NS

    @functools.partial(
        pl.kernel,
        mesh=_mesh(),
        out_type=jax.ShapeDtypeStruct((NC, npad, d), jnp.float32),
        scratch_types=[
            pltpu.VMEM((CHUNK,), jnp.int32),
            pltpu.VMEM((CHUNK,), jnp.int32),
            pltpu.VMEM((CHUNK, d), jnp.float32),
            pltpu.VMEM((CHUNK, d), jnp.float32),
            pltpu.VMEM((tail,), jnp.int32) if tail else None,
            pltpu.VMEM((tail,), jnp.int32) if tail else None,
            pltpu.VMEM_SHARED((npad, d), jnp.float32),
            pltpu.SemaphoreType.DMA,
        ],
    )
    def k(z_hbm, src_hbm, dst_hbm, out_hbm,
          src_v, dst_v, rows_v, zro_v, src_t, dst_t, acc_sh, sem):
        cid = lax.axis_index("c")
        sid = lax.axis_index("s")
        _zero_rows(zro_v, CHUNK, d)
        base = sid * rows_per_tile
        _copy_rows_spmem(zro_v, acc_sh, base, rows_per_tile)
        plsc.subcore_barrier()

        eb = cid * e_per_core + sid * e_per_tile

        @pl.loop(0, n_full)
        def _(c):
            off = eb + c * CHUNK
            pltpu.sync_copy(src_hbm.at[pl.ds(off, CHUNK)], src_v)
            pltpu.sync_copy(dst_hbm.at[pl.ds(off, CHUNK)], dst_v)
            pltpu.async_copy(z_hbm.at[src_v], rows_v, sem).wait()
            pltpu.sync_copy(rows_v, acc_sh.at[dst_v], add=True)

        if tail:
            off = eb + n_full * CHUNK
            pltpu.sync_copy(src_hbm.at[pl.ds(off, tail)], src_t)
            pltpu.sync_copy(dst_hbm.at[pl.ds(off, tail)], dst_t)
            pltpu.async_copy(z_hbm.at[src_t], rows_v.at[pl.ds(0, tail)], sem).wait()
            pltpu.sync_copy(rows_v.at[pl.ds(0, tail)], acc_sh.at[dst_t], add=True)

        plsc.subcore_barrier()
        _copy_out_spmem(acc_sh, out_hbm, cid, base, rows_per_tile)

    return k


def _norm_col(parts_ref):
    deg = jnp.sum(parts_ref[...], axis=1, keepdims=True)
    return lax.rsqrt(jnp.maximum(deg, 1.0))


def _tc_pre(x, degs_t, w1t):
    n, d = x.shape
    r = 1000

    def body(x_ref, degs_ref, w_ref, z_ref):
        norm = _norm_col(degs_ref)
        h = x_ref[...] * norm
        z_ref[...] = jnp.dot(h, w_ref[...], preferred_element_type=jnp.float32)

    return pl.pallas_call(
        body,
        grid=(n // r,),
        in_specs=[
            pl.BlockSpec((r, d), lambda i: (i, 0)),
            pl.BlockSpec((r, NW), lambda i: (i, 0)),
            pl.BlockSpec((d, d), lambda i: (0, 0)),
        ],
        out_specs=pl.BlockSpec((r, d), lambda i: (i, 0)),
        out_shape=jax.ShapeDtypeStruct((n, d), jnp.float32),
    )(x, degs_t, w1t)


def _tc_mid(agg_parts, degd_t, degs_t, b1, gamma, beta, a, w2t, n):
    d = agg_parts.shape[-1]
    r = 1000

    def body(agg_ref, degd_ref, degs_ref, b_ref, g_ref, bt_ref, a_ref, w_ref,
             z_ref):
        nd = _norm_col(degd_ref)
        h = (agg_ref[0] + agg_ref[1] + b_ref[...]) * nd
        mean = jnp.mean(h, axis=1, keepdims=True)
        var = jnp.mean((h - mean) ** 2, axis=1, keepdims=True)
        hn = (h - mean) * lax.rsqrt(var + 1e-5) * g_ref[...] + bt_ref[...]
        hp = jnp.where(hn > 0, hn, a_ref[0, 0] * hn)
        ns = _norm_col(degs_ref)
        z_ref[...] = jnp.dot(hp * ns, w_ref[...],
                             preferred_element_type=jnp.float32)

    return pl.pallas_call(
        body,
        grid=(n // r,),
        in_specs=[
            pl.BlockSpec((NC, r, d), lambda i: (0, i, 0)),
            pl.BlockSpec((r, NW), lambda i: (i, 0)),
            pl.BlockSpec((r, NW), lambda i: (i, 0)),
            pl.BlockSpec((1, d), lambda i: (0, 0)),
            pl.BlockSpec((1, d), lambda i: (0, 0)),
            pl.BlockSpec((1, d), lambda i: (0, 0)),
            pl.BlockSpec((1, 1), lambda i: (0, 0), memory_space=pltpu.SMEM),
            pl.BlockSpec((d, d), lambda i: (0, 0)),
        ],
        out_specs=pl.BlockSpec((r, d), lambda i: (i, 0)),
        out_shape=jax.ShapeDtypeStruct((n, d), jnp.float32),
    )(agg_parts, degd_t, degs_t, b1, gamma, beta, a, w2t)


def _tc_post(agg_parts, degd_t, b2, n):
    d = agg_parts.shape[-1]
    r = 1000

    def body(agg_ref, degd_ref, b_ref, o_ref):
        nd = _norm_col(degd_ref)
        o_ref[...] = (agg_ref[0] + agg_ref[1] + b_ref[...]) * nd

    return pl.pallas_call(
        body,
        grid=(n // r,),
        in_specs=[
            pl.BlockSpec((NC, r, d), lambda i: (0, i, 0)),
            pl.BlockSpec((r, NW), lambda i: (i, 0)),
            pl.BlockSpec((1, d), lambda i: (0, 0)),
        ],
        out_specs=pl.BlockSpec((r, d), lambda i: (i, 0)),
        out_shape=jax.ShapeDtypeStruct((n, d), jnp.float32),
    )(agg_parts, degd_t, b2)


def kernel(x, edge_index, W1, b1, ln_gamma, ln_beta, prelu_a, W2, b2):
    n, d = x.shape
    e = edge_index.shape[1]
    src = edge_index[0]
    dst = edge_index[1]

    degs_parts, degd_parts = _make_degree_kernel(n, e)(src, dst)
    npad = _pad_rows(n)
    degs_t = degs_parts.reshape(NW, npad)[:, :n].T
    degd_t = degd_parts.reshape(NW, npad)[:, :n].T

    segsum = _make_segsum_kernel(n, e, d)

    z1 = _tc_pre(x, degs_t, W1.T)
    agg1 = segsum(z1, src, dst)
    z2 = _tc_mid(agg1, degd_t, degs_t,
                 b1.reshape(1, d), ln_gamma.reshape(1, d),
                 ln_beta.reshape(1, d), prelu_a.reshape(1, 1), W2.T, n)
    agg2 = segsum(z2, src, dst)
    return _tc_post(agg2, degd_t, b2.reshape(1, d), n)
